# baseline (speedup 1.0000x reference)
import functools
import jax
import jax.numpy as jnp
from jax.experimental import pallas as pl
from jax.experimental.pallas import tpu as pltpu

_R = 256


def _fused_body(h_ref, mem_ref, wg_ref, ug_ref, bias_ref, out_ref):
    h = h_ref[...]
    mem = mem_ref[...]
    scores = jnp.sum(mem * h[:, None, :], axis=2)
    mx = jnp.max(scores, axis=1, keepdims=True)
    e = jnp.exp(scores - mx)
    attn = e / jnp.sum(e, axis=1, keepdims=True)
    r = jnp.sum(attn[:, :, None] * mem, axis=1)
    z = jnp.dot(h, wg_ref[...], preferred_element_type=jnp.float32)
    z = z + jnp.dot(r, ug_ref[...], preferred_element_type=jnp.float32)
    g = jax.nn.sigmoid(z + bias_ref[...])
    out_ref[...] = g * r + (1.0 - g) * h


@jax.jit
def kernel(h_tilde, mem_bank, W_g_w, W_g_b, U_g_w, U_g_b, b_g):
    B, M, d = mem_bank.shape
    wg = W_g_w.T
    ug = U_g_w.T
    bias = (W_g_b + U_g_b + b_g).reshape(1, d)
    grid = (B // _R,)
    return pl.pallas_call(
        _fused_body,
        grid=grid,
        in_specs=[
            pl.BlockSpec((_R, d), lambda i: (i, 0)),
            pl.BlockSpec((_R, M, d), lambda i: (i, 0, 0)),
            pl.BlockSpec((d, d), lambda i: (0, 0)),
            pl.BlockSpec((d, d), lambda i: (0, 0)),
            pl.BlockSpec((1, d), lambda i: (0, 0)),
        ],
        out_specs=pl.BlockSpec((_R, d), lambda i: (i, 0)),
        out_shape=jax.ShapeDtypeStruct((B, d), jnp.float32),
        compiler_params=pltpu.CompilerParams(
            dimension_semantics=("arbitrary",),
        ),
    )(h_tilde, mem_bank, wg, ug, bias)
